# Initial kernel scaffold; baseline (speedup 1.0000x reference)
import jax
import jax.numpy as jnp
from jax.experimental import pallas as pl


def kernel(indices, weights, table):
    raise NotImplementedError("write your pallas kernel here")


# speedup vs baseline: 1.9603x; 1.9603x over previous
import functools

import jax
import jax.numpy as jnp
from jax import lax
from jax.experimental import pallas as pl
from jax.experimental.pallas import tpu as pltpu
from jax.experimental.pallas import tpu_sc as plsc

NC = 2
NS = 16
NW = NC * NS
LANES = 16

CHUNK = 128
RING = 8


def _rowsum_tc(table):
    v, d = table.shape
    blk = 8000
    assert v % blk == 0
    nblk = v // blk
    t3 = table.reshape(nblk, blk, d)

    def body(t_ref, o_ref):
        o_ref[...] = jnp.sum(t_ref[...], axis=-1)[None]

    out = pl.pallas_call(
        body,
        grid=(nblk,),
        in_specs=[pl.BlockSpec((1, blk, d), lambda i: (i, 0, 0))],
        out_specs=pl.BlockSpec((1, 1, blk), lambda i: (i, 0, 0)),
        out_shape=jax.ShapeDtypeStruct((nblk, 1, blk), jnp.float32),
    )(t3)
    return out.reshape(v)


def _gather_mul_sc(idx2, w2, rowsum):
    n_rows = idx2.shape[0]
    assert n_rows % NW == 0
    n_ch = n_rows // NW

    mesh = plsc.VectorSubcoreMesh(core_axis_name="c", subcore_axis_name="s")

    @functools.partial(
        pl.kernel,
        mesh=mesh,
        out_type=jax.ShapeDtypeStruct((n_rows, CHUNK), jnp.float32),
        scratch_types=[
            pltpu.VMEM((n_ch, CHUNK), jnp.int32),
            pltpu.VMEM((n_ch, CHUNK), jnp.float32),
            pltpu.VMEM((n_ch, CHUNK), jnp.float32),
            pltpu.VMEM((n_ch, CHUNK), jnp.float32),
            pltpu.SemaphoreType.DMA,
        ],
    )
    def k(idx_hbm, w_hbm, rs_hbm, out_hbm, idx_v, w_v, val_v, out_v, sem):
        wid = lax.axis_index("s") * NC + lax.axis_index("c")
        row0 = wid * n_ch
        pltpu.sync_copy(idx_hbm.at[pl.ds(row0, n_ch)], idx_v)
        pltpu.sync_copy(w_hbm.at[pl.ds(row0, n_ch)], w_v)

        def start(c):
            pltpu.make_async_copy(
                rs_hbm.at[idx_v.at[c]], val_v.at[c], sem).start()

        def wait(c):
            pltpu.make_async_copy(
                rs_hbm.at[idx_v.at[c]], val_v.at[c], sem).wait()

        def prime(c, carry):
            start(c)
            return carry
        lax.fori_loop(0, RING, prime, 0)

        def step(c, carry):
            start(c + RING)
            wait(c)
            return carry
        lax.fori_loop(0, n_ch - RING, step, 0)

        def drain(c, carry):
            wait(c)
            return carry
        lax.fori_loop(n_ch - RING, n_ch, drain, 0)

        def mul(c, carry):
            for j in range(CHUNK // LANES):
                sl = pl.ds(j * LANES, LANES)
                out_v[c, sl] = val_v[c, sl] * w_v[c, sl]
            return carry
        lax.fori_loop(0, n_ch, mul, 0)

        pltpu.sync_copy(out_v, out_hbm.at[pl.ds(row0, n_ch)])

    return k(idx2, w2, rowsum)


def kernel(indices, weights, table):
    b, s, n = indices.shape
    tot = b * s * n
    assert tot % (NW * CHUNK) == 0
    idx2 = indices.reshape(tot // CHUNK, CHUNK).astype(jnp.int32)
    w2 = weights.reshape(tot // CHUNK, CHUNK)
    rowsum = _rowsum_tc(table)
    out = _gather_mul_sc(idx2, w2, rowsum)
    return out.reshape(b, s)
